# baseline (speedup 1.0000x reference)
import functools

import jax
import jax.numpy as jnp
from jax import lax
from jax.experimental import pallas as pl
from jax.experimental.pallas import tpu as pltpu
from jax.experimental.pallas import tpu_sc as plsc

N = 10000
E = 320000
D_NODE = 128
D_EDGE = 16
D = 128

CHUNK = 80
T_EDGES = E // 16
TCH = T_EDGES // CHUNK
SUP = 25
NSUP = TCH // SUP
CAP = T_EDGES + 2 * CHUNK
HALF = N // 2
ACC_ROWS = HALF + 8

_LOG2 = 0.6931471805599453


def _ssp(x):
    return jnp.log1p(jnp.exp(x)) - _LOG2


def _nodeproj_body(x_ref, w_ref, b_ref, o_ref):
    o_ref[...] = jnp.dot(x_ref[...], w_ref[...],
                         preferred_element_type=jnp.float32) + b_ref[...]


def _node_proj(x, w, b2d):
    blk = 1000
    return pl.pallas_call(
        _nodeproj_body,
        grid=(N // blk,),
        in_specs=[
            pl.BlockSpec((blk, D), lambda i: (i, 0)),
            pl.BlockSpec((D, D), lambda i: (0, 0)),
            pl.BlockSpec((1, D), lambda i: (0, 0)),
        ],
        out_specs=pl.BlockSpec((blk, D), lambda i: (i, 0)),
        out_shape=jax.ShapeDtypeStruct((N, D), jnp.float32),
    )(x, w, b2d)


def _edge_body(e_ref, wa_ref, ba_ref, wb_ref, bb_ref, he_ref):
    e = e_ref[...]
    h1 = _ssp(jnp.dot(e, wa_ref[...], preferred_element_type=jnp.float32)
              + ba_ref[...])
    he_ref[...] = _ssp(jnp.dot(h1, wb_ref[...],
                               preferred_element_type=jnp.float32)
                       + bb_ref[...])


def _edge_mlp(e, wa, ba, wb, bb):
    blk = 2000
    return pl.pallas_call(
        _edge_body,
        grid=(E // blk,),
        in_specs=[
            pl.BlockSpec((blk, D_EDGE), lambda i: (i, 0)),
            pl.BlockSpec((D_EDGE, D), lambda i: (0, 0)),
            pl.BlockSpec((1, D), lambda i: (0, 0)),
            pl.BlockSpec((D, D), lambda i: (0, 0)),
            pl.BlockSpec((1, D), lambda i: (0, 0)),
        ],
        out_specs=pl.BlockSpec((blk, D), lambda i: (i, 0)),
        out_shape=jax.ShapeDtypeStruct((E, D), jnp.float32),
    )(e, wa, ba, wb, bb)


def _mid_body(p_ref, wo_ref, bo_ref, wn_ref, bn_ref, o_ref):
    agg = p_ref[0]
    t = jnp.tanh(_ssp(jnp.dot(agg, wo_ref[...],
                              preferred_element_type=jnp.float32)
                      + bo_ref[...]))
    o_ref[...] = jnp.dot(t, wn_ref[...],
                         preferred_element_type=jnp.float32) + bn_ref[...]


def _mid_proj(p, wo, bo2d, wn, bn2d):
    blk = 1000
    wspec = pl.BlockSpec((D, D), lambda i: (0, 0))
    bspec = pl.BlockSpec((1, D), lambda i: (0, 0))
    return pl.pallas_call(
        _mid_body,
        grid=(N // blk,),
        in_specs=[
            pl.BlockSpec((1, blk, D), lambda i: (i // 5, i % 5, 0)),
            wspec, bspec, wspec, bspec,
        ],
        out_specs=pl.BlockSpec((blk, D), lambda i: (i, 0)),
        out_shape=jax.ShapeDtypeStruct((N, D), jnp.float32),
    )(p, wo, bo2d, wn, bn2d)


def _fin_body(p_ref, wo_ref, bo_ref, o_ref):
    agg = p_ref[0]
    o_ref[...] = jnp.tanh(_ssp(jnp.dot(agg, wo_ref[...],
                                       preferred_element_type=jnp.float32)
                               + bo_ref[...]))


def _fin_proj(p, wo, bo2d):
    blk = 1000
    return pl.pallas_call(
        _fin_body,
        grid=(N // blk,),
        in_specs=[
            pl.BlockSpec((1, blk, D), lambda i: (i // 5, i % 5, 0)),
            pl.BlockSpec((D, D), lambda i: (0, 0)),
            pl.BlockSpec((1, D), lambda i: (0, 0)),
        ],
        out_specs=pl.BlockSpec((blk, D), lambda i: (i, 0)),
        out_shape=jax.ShapeDtypeStruct((N, D), jnp.float32),
    )(p, wo, bo2d)


def _sc_body(hv_hbm, he_hbm, src_hbm, dst_hbm, out_hbm,
             src_v, tidx0, tidx1, tidx2, dbuf0, dbuf1, dbuf2,
             gbuf0, gbuf1, gbuf2, hbuf0, hbuf1, hbuf2, acc,
             gsem0, gsem1, gsem2, hsem0, hsem1, hsem2,
             dsem0, dsem1, dsem2, ssem0, ssem1, ssem2):
    c = lax.axis_index("c")
    s = lax.axis_index("s")
    base = c * HALF
    gbufs = (gbuf0, gbuf1, gbuf2)
    hbufs = (hbuf0, hbuf1, hbuf2)
    dbufs = (dbuf0, dbuf1, dbuf2)
    tidxs = (tidx0, tidx1, tidx2)
    gsems = (gsem0, gsem1, gsem2)
    hsems = (hsem0, hsem1, hsem2)
    dsems = (dsem0, dsem1, dsem2)
    ssems = (ssem0, ssem1, ssem2)

    pltpu.sync_copy(src_hbm.at[pl.ds(s * T_EDGES, T_EDGES)], src_v)

    def zbody(r, carry):
        for t in range(8):
            gbuf0[r, pl.ds(t * 16, 16)] = jnp.zeros((16,), jnp.float32)
        return carry
    lax.fori_loop(0, CHUNK, zbody, 0)
    for g in range(63):
        sz = CHUNK if g < 62 else ACC_ROWS - 62 * CHUNK

        @pl.when(g % 16 == s)
        def _():
            pltpu.sync_copy(gbuf0.at[pl.ds(0, sz)],
                            acc.at[pl.ds(g * CHUNK, sz)])
    plsc.subcore_barrier()

    def fire(chunk, b):
        pltpu.async_copy(hv_hbm.at[src_v.at[pl.ds(chunk * CHUNK, CHUNK)]],
                         gbufs[b], gsems[b])
        pltpu.async_copy(
            he_hbm.at[pl.ds(s * T_EDGES + chunk * CHUNK, CHUNK)],
            hbufs[b], hsems[b])
        pltpu.async_copy(
            dst_hbm.at[pl.ds(s * T_EDGES + chunk * CHUNK, CHUNK)],
            dbufs[b], dsems[b])

    def wait_in(chunk, b):
        pltpu.make_async_copy(hv_hbm.at[pl.ds(0, CHUNK)],
                              gbufs[b], gsems[b]).wait()
        pltpu.make_async_copy(he_hbm.at[pl.ds(0, CHUNK)],
                              hbufs[b], hsems[b]).wait()
        pltpu.make_async_copy(dst_hbm.at[pl.ds(0, CHUNK)],
                              dbufs[b], dsems[b]).wait()
        for k in range(5):
            sl = pl.ds(k * 16, 16)
            d = dbufs[b][sl] - base
            oob = (d < 0) | (d >= HALF)
            tidxs[b][sl] = jnp.where(oob, HALF + (d & 7), d)

        def mbody(r, inner):
            for t in range(8):
                sl = pl.ds(t * 16, 16)
                gbufs[b][r, sl] = gbufs[b][r, sl] * hbufs[b][r, sl]
            return inner
        lax.fori_loop(0, CHUNK, mbody, 0)

    def wait_scat(b):
        pltpu.make_async_copy(gbufs[b], acc.at[tidxs[b]], ssems[b]).wait()

    def scat(b):
        pltpu.async_copy(gbufs[b], acc.at[tidxs[b]], ssems[b], add=True)

    fire(0, 0)
    fire(1, 1)

    def group_body(g3, carry):
        chunk = g3 * 3
        for p in range(3):
            b = p
            bn = (p + 2) % 3
            wait_in(chunk + p, b)

            @pl.when(chunk + p >= 1)
            def _():
                wait_scat(bn)

            @pl.when(chunk + p + 2 < TCH)
            def _():
                fire(chunk + p + 2, bn)
            scat(b)
        return carry
    lax.fori_loop(0, TCH // 3, group_body, 0)

    wait_in(TCH - 1, 0)
    wait_scat(2)
    scat(0)
    wait_scat(0)

    plsc.subcore_barrier()
    for g in range(63):
        sz = CHUNK if g < 62 else HALF - 62 * CHUNK

        @pl.when(g % 16 == s)
        def _():
            pltpu.sync_copy(acc.at[pl.ds(g * CHUNK, sz)],
                            gbuf0.at[pl.ds(0, sz)])
            pltpu.sync_copy(gbuf0.at[pl.ds(0, sz)],
                            out_hbm.at[c, pl.ds(g * CHUNK, sz)])


@functools.cache
def _make_sc_gms():
    return pl.kernel(
        _sc_body,
        out_type=jax.ShapeDtypeStruct((2, HALF, D), jnp.float32),
        mesh=plsc.VectorSubcoreMesh(core_axis_name="c", subcore_axis_name="s"),
        scratch_types=[
            pltpu.VMEM((T_EDGES,), jnp.int32),
            pltpu.VMEM((CHUNK,), jnp.int32),
            pltpu.VMEM((CHUNK,), jnp.int32),
            pltpu.VMEM((CHUNK,), jnp.int32),
            pltpu.VMEM((CHUNK,), jnp.int32),
            pltpu.VMEM((CHUNK,), jnp.int32),
            pltpu.VMEM((CHUNK,), jnp.int32),
            pltpu.VMEM((CHUNK, D), jnp.float32),
            pltpu.VMEM((CHUNK, D), jnp.float32),
            pltpu.VMEM((CHUNK, D), jnp.float32),
            pltpu.VMEM((CHUNK, D), jnp.float32),
            pltpu.VMEM((CHUNK, D), jnp.float32),
            pltpu.VMEM((CHUNK, D), jnp.float32),
            pltpu.VMEM_SHARED((ACC_ROWS, D), jnp.float32),
        ] + [pltpu.SemaphoreType.DMA] * 12,
    )


def _sc_gms(hv, he, src, dst):
    return _make_sc_gms()(hv, he, src, dst)


def kernel(node_inputs, edge_inputs, edge_index,
           Wn1, bn1, We1a, be1a, We1b, be1b, Wo1, bo1,
           Wn2, bn2, We2a, be2a, We2b, be2b, Wo2, bo2):
    src = edge_index[0]
    dst = edge_index[1]

    bn1_2 = bn1.reshape(1, D)
    be1a_2 = be1a.reshape(1, D)
    be1b_2 = be1b.reshape(1, D)
    bo1_2 = bo1.reshape(1, D)
    bn2_2 = bn2.reshape(1, D)
    be2a_2 = be2a.reshape(1, D)
    be2b_2 = be2b.reshape(1, D)
    bo2_2 = bo2.reshape(1, D)

    he1 = _edge_mlp(edge_inputs, We1a, be1a_2, We1b, be1b_2)
    hv1 = _node_proj(node_inputs, Wn1, bn1_2)
    p1 = _sc_gms(hv1, he1, src, dst)
    he2 = _edge_mlp(edge_inputs, We2a, be2a_2, We2b, be2b_2)
    hv2 = _mid_proj(p1, Wo1, bo1_2, Wn2, bn2_2)
    p2 = _sc_gms(hv2, he2, src, dst)
    return _fin_proj(p2, Wo2, bo2_2)


# speedup vs baseline: 1.3016x; 1.3016x over previous
import functools

import jax
import jax.numpy as jnp
from jax import lax
from jax.experimental import pallas as pl
from jax.experimental.pallas import tpu as pltpu
from jax.experimental.pallas import tpu_sc as plsc

N = 10000
E = 320000
D_NODE = 128
D_EDGE = 16
D = 128

CHUNK = 80
T_EDGES = E // 16
T_PAIRS = T_EDGES // 2
PCH = T_PAIRS // CHUNK
E_PAIRS = E // 2
HALF = N // 2
ACC_ROWS = HALF + 8

_LOG2 = 0.6931471805599453


def _ssp(x):
    return jnp.log1p(jnp.exp(x)) - _LOG2


def _nodeproj_body(x_ref, w_ref, b_ref, o_ref):
    o_ref[...] = jnp.dot(x_ref[...], w_ref[...],
                         preferred_element_type=jnp.float32) + b_ref[...]


def _node_proj(x, w, b2d):
    blk = 1000
    return pl.pallas_call(
        _nodeproj_body,
        grid=(N // blk,),
        in_specs=[
            pl.BlockSpec((blk, D), lambda i: (i, 0)),
            pl.BlockSpec((D, D), lambda i: (0, 0)),
            pl.BlockSpec((1, D), lambda i: (0, 0)),
        ],
        out_specs=pl.BlockSpec((blk, D), lambda i: (i, 0)),
        out_shape=jax.ShapeDtypeStruct((N, D), jnp.float32),
    )(x, w, b2d)


def _rne16(x):
    u = jax.lax.bitcast_convert_type(x, jnp.uint32)
    return (u + 0x7FFF + ((u >> 16) & 1)) >> 16


def _edge_body(ea_ref, eb_ref, wa_ref, ba_ref, wb_ref, bb_ref, hp_ref):
    def he_of(et):
        h1 = _ssp(lax.dot_general(et, wa_ref[...], (((0,), (0,)), ((), ())),
                                  preferred_element_type=jnp.float32)
                  + ba_ref[...])
        return _ssp(jnp.dot(h1, wb_ref[...],
                            preferred_element_type=jnp.float32) + bb_ref[...])
    lo = _rne16(he_of(ea_ref[...]))
    hi = _rne16(he_of(eb_ref[...]))
    hp_ref[...] = ((hi << 16) | lo).astype(jnp.int32)


def _edge_mlp(e, wa, ba, wb, bb):
    blk = 1280
    return pl.pallas_call(
        _edge_body,
        grid=(E_PAIRS // blk,),
        in_specs=[
            pl.BlockSpec((D_EDGE, blk), lambda g: (0, g)),
            pl.BlockSpec((D_EDGE, blk), lambda g: (0, g + E_PAIRS // blk)),
            pl.BlockSpec((D_EDGE, D), lambda g: (0, 0)),
            pl.BlockSpec((1, D), lambda g: (0, 0)),
            pl.BlockSpec((D, D), lambda g: (0, 0)),
            pl.BlockSpec((1, D), lambda g: (0, 0)),
        ],
        out_specs=pl.BlockSpec((blk, D), lambda g: (g, 0)),
        out_shape=jax.ShapeDtypeStruct((E_PAIRS, D), jnp.int32),
    )(e, e, wa, ba, wb, bb)


def _mid_body(p_ref, wo_ref, bo_ref, wn_ref, bn_ref, o_ref):
    agg = p_ref[0]
    t = jnp.tanh(_ssp(jnp.dot(agg, wo_ref[...],
                              preferred_element_type=jnp.float32)
                      + bo_ref[...]))
    o_ref[...] = jnp.dot(t, wn_ref[...],
                         preferred_element_type=jnp.float32) + bn_ref[...]


def _mid_proj(p, wo, bo2d, wn, bn2d):
    blk = 1000
    wspec = pl.BlockSpec((D, D), lambda i: (0, 0))
    bspec = pl.BlockSpec((1, D), lambda i: (0, 0))
    return pl.pallas_call(
        _mid_body,
        grid=(N // blk,),
        in_specs=[
            pl.BlockSpec((1, blk, D), lambda i: (i // 5, i % 5, 0)),
            wspec, bspec, wspec, bspec,
        ],
        out_specs=pl.BlockSpec((blk, D), lambda i: (i, 0)),
        out_shape=jax.ShapeDtypeStruct((N, D), jnp.float32),
    )(p, wo, bo2d, wn, bn2d)


def _fin_body(p_ref, wo_ref, bo_ref, o_ref):
    agg = p_ref[0]
    o_ref[...] = jnp.tanh(_ssp(jnp.dot(agg, wo_ref[...],
                                       preferred_element_type=jnp.float32)
                               + bo_ref[...]))


def _fin_proj(p, wo, bo2d):
    blk = 1000
    return pl.pallas_call(
        _fin_body,
        grid=(N // blk,),
        in_specs=[
            pl.BlockSpec((1, blk, D), lambda i: (i // 5, i % 5, 0)),
            pl.BlockSpec((D, D), lambda i: (0, 0)),
            pl.BlockSpec((1, D), lambda i: (0, 0)),
        ],
        out_specs=pl.BlockSpec((blk, D), lambda i: (i, 0)),
        out_shape=jax.ShapeDtypeStruct((N, D), jnp.float32),
    )(p, wo, bo2d)


def _sc_body(hv_hbm, hp_hbm, src_hbm, dst_hbm, out_hbm,
             src_v, tidxa, tidxb, dbuf0, dbuf1,
             ga0, ga1, gb0, gb1, hp0, hp1, acc,
             sa0, sa1, sb0, sb1, sh0, sh1, sd0, sd1):
    c = lax.axis_index("c")
    s = lax.axis_index("s")
    base = c * HALF
    gas = (ga0, ga1)
    gbs = (gb0, gb1)
    hps = (hp0, hp1)
    dbufs = (dbuf0, dbuf1)
    sas = (sa0, sa1)
    sbs = (sb0, sb1)
    shs = (sh0, sh1)
    sds = (sd0, sd1)

    pltpu.sync_copy(src_hbm.at[pl.ds(s * T_PAIRS, T_PAIRS)],
                    src_v.at[pl.ds(0, T_PAIRS)])
    pltpu.sync_copy(src_hbm.at[pl.ds(E // 2 + s * T_PAIRS, T_PAIRS)],
                    src_v.at[pl.ds(T_PAIRS, T_PAIRS)])

    def zbody(r, carry):
        for t in range(8):
            ga0[r, pl.ds(t * 16, 16)] = jnp.zeros((16,), jnp.float32)
        return carry
    lax.fori_loop(0, CHUNK, zbody, 0)
    for g in range(63):
        sz = CHUNK if g < 62 else ACC_ROWS - 62 * CHUNK

        @pl.when(g % 16 == s)
        def _():
            pltpu.sync_copy(ga0.at[pl.ds(0, sz)],
                            acc.at[pl.ds(g * CHUNK, sz)])
    plsc.subcore_barrier()

    def fire(chunk, b):
        lo0 = s * T_PAIRS + chunk * CHUNK
        pltpu.async_copy(hv_hbm.at[src_v.at[pl.ds(chunk * CHUNK, CHUNK)]],
                         gas[b], sas[b])
        pltpu.async_copy(
            hv_hbm.at[src_v.at[pl.ds(T_PAIRS + chunk * CHUNK, CHUNK)]],
            gbs[b], sbs[b])
        pltpu.async_copy(
            hp_hbm.at[pl.ds(s * T_PAIRS + chunk * CHUNK, CHUNK)],
            hps[b], shs[b])
        pltpu.async_copy(dst_hbm.at[pl.ds(lo0, CHUNK)],
                         dbufs[b].at[pl.ds(0, CHUNK)], sds[b])
        pltpu.async_copy(dst_hbm.at[pl.ds(E // 2 + lo0, CHUNK)],
                         dbufs[b].at[pl.ds(CHUNK, CHUNK)], sds[b])

    def wait_in(b):
        pltpu.make_async_copy(hv_hbm.at[pl.ds(0, CHUNK)],
                              gas[b], sas[b]).wait()
        pltpu.make_async_copy(hv_hbm.at[pl.ds(0, CHUNK)],
                              gbs[b], sbs[b]).wait()
        pltpu.make_async_copy(hp_hbm.at[pl.ds(0, CHUNK)],
                              hps[b], shs[b]).wait()
        pltpu.make_async_copy(dst_hbm.at[pl.ds(0, CHUNK)],
                              dbufs[b].at[pl.ds(0, CHUNK)], sds[b]).wait()
        pltpu.make_async_copy(dst_hbm.at[pl.ds(0, CHUNK)],
                              dbufs[b].at[pl.ds(CHUNK, CHUNK)],
                              sds[b]).wait()
        for k in range(5):
            sl = pl.ds(k * 16, 16)
            da = dbufs[b][sl] - base
            ooba = (da < 0) | (da >= HALF)
            tidxa[sl] = jnp.where(ooba, HALF + (da & 7), da)
            db = dbufs[b][pl.ds(CHUNK + k * 16, 16)] - base
            oobb = (db < 0) | (db >= HALF)
            tidxb[sl] = jnp.where(oobb, HALF + (db & 7), db)

        @plsc.parallel_loop(0, CHUNK, 1, unroll=2)
        def mbody(r):
            for t in range(8):
                sl = pl.ds(t * 16, 16)
                w = hps[b][r, sl]
                lo = plsc.bitcast(w << 16, jnp.float32)
                hi = plsc.bitcast(w & jnp.int32(-65536), jnp.float32)
                gas[b][r, sl] = gas[b][r, sl] * lo
                gbs[b][r, sl] = gbs[b][r, sl] * hi
        pltpu.sync_copy(gas[b], acc.at[tidxa], add=True)
        pltpu.sync_copy(gbs[b], acc.at[tidxb], add=True)

    fire(0, 0)
    fire(1, 1)

    def pair_body(j2, carry):
        for b in (0, 1):
            chunk = j2 * 2 + b
            wait_in(b)

            @pl.when(chunk + 2 < PCH)
            def _():
                fire(chunk + 2, b)
        return carry
    lax.fori_loop(0, PCH // 2, pair_body, 0)

    wait_in(0)

    plsc.subcore_barrier()
    for g in range(63):
        sz = CHUNK if g < 62 else HALF - 62 * CHUNK

        @pl.when(g % 16 == s)
        def _():
            pltpu.sync_copy(acc.at[pl.ds(g * CHUNK, sz)],
                            ga0.at[pl.ds(0, sz)])
            pltpu.sync_copy(ga0.at[pl.ds(0, sz)],
                            out_hbm.at[c, pl.ds(g * CHUNK, sz)])


@functools.cache
def _make_sc_gms():
    return pl.kernel(
        _sc_body,
        out_type=jax.ShapeDtypeStruct((2, HALF, D), jnp.float32),
        mesh=plsc.VectorSubcoreMesh(core_axis_name="c", subcore_axis_name="s"),
        compiler_params=pltpu.CompilerParams(needs_layout_passes=False),
        scratch_types=[
            pltpu.VMEM((T_EDGES,), jnp.int32),
            pltpu.VMEM((CHUNK,), jnp.int32),
            pltpu.VMEM((CHUNK,), jnp.int32),
            pltpu.VMEM((2 * CHUNK,), jnp.int32),
            pltpu.VMEM((2 * CHUNK,), jnp.int32),
            pltpu.VMEM((CHUNK, D), jnp.float32),
            pltpu.VMEM((CHUNK, D), jnp.float32),
            pltpu.VMEM((CHUNK, D), jnp.float32),
            pltpu.VMEM((CHUNK, D), jnp.float32),
            pltpu.VMEM((CHUNK, D), jnp.int32),
            pltpu.VMEM((CHUNK, D), jnp.int32),
            pltpu.VMEM_SHARED((ACC_ROWS, D), jnp.float32),
        ] + [pltpu.SemaphoreType.DMA] * 8,
    )


def _sc_gms(hv, he, src, dst):
    return _make_sc_gms()(hv, he, src, dst)


def kernel(node_inputs, edge_inputs, edge_index,
           Wn1, bn1, We1a, be1a, We1b, be1b, Wo1, bo1,
           Wn2, bn2, We2a, be2a, We2b, be2b, Wo2, bo2):
    src = edge_index[0]
    dst = edge_index[1]

    bn1_2 = bn1.reshape(1, D)
    be1a_2 = be1a.reshape(1, D)
    be1b_2 = be1b.reshape(1, D)
    bo1_2 = bo1.reshape(1, D)
    bn2_2 = bn2.reshape(1, D)
    be2a_2 = be2a.reshape(1, D)
    be2b_2 = be2b.reshape(1, D)
    bo2_2 = bo2.reshape(1, D)

    e_t = edge_inputs.T
    he1 = _edge_mlp(e_t, We1a, be1a_2, We1b, be1b_2)
    hv1 = _node_proj(node_inputs, Wn1, bn1_2)
    p1 = _sc_gms(hv1, he1, src, dst)
    he2 = _edge_mlp(e_t, We2a, be2a_2, We2b, be2b_2)
    hv2 = _mid_proj(p1, Wo1, bo1_2, Wn2, bn2_2)
    p2 = _sc_gms(hv2, he2, src, dst)
    return _fin_proj(p2, Wo2, bo2_2)


# speedup vs baseline: 1.3261x; 1.0189x over previous
import functools

import jax
import jax.numpy as jnp
from jax import lax
from jax.experimental import pallas as pl
from jax.experimental.pallas import tpu as pltpu
from jax.experimental.pallas import tpu_sc as plsc

N = 10000
E = 320000
D_NODE = 128
D_EDGE = 16
D = 128

CHUNK = 80
T_EDGES = E // 16
T_PAIRS = T_EDGES // 2
PCH = T_PAIRS // CHUNK
E_PAIRS = E // 2
HALF = N // 2
ACC_ROWS = HALF + 8

_LOG2 = 0.6931471805599453


def _ssp(x):
    return jnp.log1p(jnp.exp(x)) - _LOG2


def _nodeproj_body(x_ref, w_ref, b_ref, o_ref):
    o_ref[...] = jnp.dot(x_ref[...], w_ref[...],
                         preferred_element_type=jnp.float32) + b_ref[...]


def _node_proj(x, w, b2d):
    blk = 1000
    return pl.pallas_call(
        _nodeproj_body,
        grid=(N // blk,),
        in_specs=[
            pl.BlockSpec((blk, D), lambda i: (i, 0)),
            pl.BlockSpec((D, D), lambda i: (0, 0)),
            pl.BlockSpec((1, D), lambda i: (0, 0)),
        ],
        out_specs=pl.BlockSpec((blk, D), lambda i: (i, 0)),
        out_shape=jax.ShapeDtypeStruct((N, D), jnp.float32),
    )(x, w, b2d)


def _rne16(x):
    u = jax.lax.bitcast_convert_type(x, jnp.uint32)
    return (u + 0x7FFF + ((u >> 16) & 1)) >> 16


def _edge_body(ea_ref, eb_ref, wa_ref, ba_ref, wb_ref, bb_ref, hp_ref):
    def he_of(et):
        h1 = _ssp(lax.dot_general(et, wa_ref[...], (((0,), (0,)), ((), ())),
                                  preferred_element_type=jnp.float32)
                  + ba_ref[...])
        return _ssp(jnp.dot(h1, wb_ref[...],
                            preferred_element_type=jnp.float32) + bb_ref[...])
    lo = _rne16(he_of(ea_ref[...]))
    hi = _rne16(he_of(eb_ref[...]))
    hp_ref[...] = ((hi << 16) | lo).astype(jnp.int32)


def _edge_mlp(e, wa, ba, wb, bb):
    blk = 1280
    return pl.pallas_call(
        _edge_body,
        grid=(E_PAIRS // blk,),
        in_specs=[
            pl.BlockSpec((D_EDGE, blk), lambda g: (0, g)),
            pl.BlockSpec((D_EDGE, blk), lambda g: (0, g + E_PAIRS // blk)),
            pl.BlockSpec((D_EDGE, D), lambda g: (0, 0)),
            pl.BlockSpec((1, D), lambda g: (0, 0)),
            pl.BlockSpec((D, D), lambda g: (0, 0)),
            pl.BlockSpec((1, D), lambda g: (0, 0)),
        ],
        out_specs=pl.BlockSpec((blk, D), lambda g: (g, 0)),
        out_shape=jax.ShapeDtypeStruct((E_PAIRS, D), jnp.int32),
    )(e, e, wa, ba, wb, bb)


def _mid_body(p_ref, wo_ref, bo_ref, wn_ref, bn_ref, o_ref):
    agg = p_ref[0]
    t = jnp.tanh(_ssp(jnp.dot(agg, wo_ref[...],
                              preferred_element_type=jnp.float32)
                      + bo_ref[...]))
    o_ref[...] = jnp.dot(t, wn_ref[...],
                         preferred_element_type=jnp.float32) + bn_ref[...]


def _mid_proj(p, wo, bo2d, wn, bn2d):
    blk = 1000
    wspec = pl.BlockSpec((D, D), lambda i: (0, 0))
    bspec = pl.BlockSpec((1, D), lambda i: (0, 0))
    return pl.pallas_call(
        _mid_body,
        grid=(N // blk,),
        in_specs=[
            pl.BlockSpec((1, blk, D), lambda i: (i // 5, i % 5, 0)),
            wspec, bspec, wspec, bspec,
        ],
        out_specs=pl.BlockSpec((blk, D), lambda i: (i, 0)),
        out_shape=jax.ShapeDtypeStruct((N, D), jnp.float32),
    )(p, wo, bo2d, wn, bn2d)


def _fin_body(p_ref, wo_ref, bo_ref, o_ref):
    agg = p_ref[0]
    o_ref[...] = jnp.tanh(_ssp(jnp.dot(agg, wo_ref[...],
                                       preferred_element_type=jnp.float32)
                               + bo_ref[...]))


def _fin_proj(p, wo, bo2d):
    blk = 1000
    return pl.pallas_call(
        _fin_body,
        grid=(N // blk,),
        in_specs=[
            pl.BlockSpec((1, blk, D), lambda i: (i // 5, i % 5, 0)),
            pl.BlockSpec((D, D), lambda i: (0, 0)),
            pl.BlockSpec((1, D), lambda i: (0, 0)),
        ],
        out_specs=pl.BlockSpec((blk, D), lambda i: (i, 0)),
        out_shape=jax.ShapeDtypeStruct((N, D), jnp.float32),
    )(p, wo, bo2d)


def _sc_body(hv_hbm, hp_hbm, src_hbm, dst_hbm, out_hbm,
             src_v, tidxa, tidxb, dbuf0, dbuf1,
             ga0, ga1, gb0, gb1, hp0, hp1, acc,
             sa0, sa1, sb0, sb1, sh0, sh1, sd0, sd1):
    c = lax.axis_index("c")
    s = lax.axis_index("s")
    base = c * HALF
    gas = (ga0, ga1)
    gbs = (gb0, gb1)
    hps = (hp0, hp1)
    dbufs = (dbuf0, dbuf1)
    sas = (sa0, sa1)
    sbs = (sb0, sb1)
    shs = (sh0, sh1)
    sds = (sd0, sd1)

    pltpu.sync_copy(src_hbm.at[pl.ds(s * T_PAIRS, T_PAIRS)],
                    src_v.at[pl.ds(0, T_PAIRS)])
    pltpu.sync_copy(src_hbm.at[pl.ds(E // 2 + s * T_PAIRS, T_PAIRS)],
                    src_v.at[pl.ds(T_PAIRS, T_PAIRS)])

    def zbody(r, carry):
        for t in range(8):
            ga0[r, pl.ds(t * 16, 16)] = jnp.zeros((16,), jnp.float32)
        return carry
    lax.fori_loop(0, CHUNK, zbody, 0)
    for g in range(63):
        sz = CHUNK if g < 62 else ACC_ROWS - 62 * CHUNK

        @pl.when(g % 16 == s)
        def _():
            pltpu.sync_copy(ga0.at[pl.ds(0, sz)],
                            acc.at[pl.ds(g * CHUNK, sz)])
    plsc.subcore_barrier()

    def fire(chunk, b):
        lo0 = s * T_PAIRS + chunk * CHUNK
        pltpu.async_copy(hv_hbm.at[src_v.at[pl.ds(chunk * CHUNK, CHUNK)]],
                         gas[b], sas[b])
        pltpu.async_copy(
            hv_hbm.at[src_v.at[pl.ds(T_PAIRS + chunk * CHUNK, CHUNK)]],
            gbs[b], sbs[b])
        pltpu.async_copy(
            hp_hbm.at[pl.ds(s * T_PAIRS + chunk * CHUNK, CHUNK)],
            hps[b], shs[b])
        pltpu.async_copy(dst_hbm.at[pl.ds(lo0, CHUNK)],
                         dbufs[b].at[pl.ds(0, CHUNK)], sds[b])
        pltpu.async_copy(dst_hbm.at[pl.ds(E // 2 + lo0, CHUNK)],
                         dbufs[b].at[pl.ds(CHUNK, CHUNK)], sds[b])

    def wait_in(b):
        pltpu.make_async_copy(hv_hbm.at[pl.ds(0, CHUNK)],
                              gas[b], sas[b]).wait()
        pltpu.make_async_copy(hv_hbm.at[pl.ds(0, CHUNK)],
                              gbs[b], sbs[b]).wait()
        pltpu.make_async_copy(hp_hbm.at[pl.ds(0, CHUNK)],
                              hps[b], shs[b]).wait()
        pltpu.make_async_copy(dst_hbm.at[pl.ds(0, CHUNK)],
                              dbufs[b].at[pl.ds(0, CHUNK)], sds[b]).wait()
        pltpu.make_async_copy(dst_hbm.at[pl.ds(0, CHUNK)],
                              dbufs[b].at[pl.ds(CHUNK, CHUNK)],
                              sds[b]).wait()
        for k in range(5):
            sl = pl.ds(k * 16, 16)
            da = dbufs[b][sl] - base
            ooba = (da < 0) | (da >= HALF)
            tidxa[sl] = jnp.where(ooba, HALF + (da & 7), da)
            db = dbufs[b][pl.ds(CHUNK + k * 16, 16)] - base
            oobb = (db < 0) | (db >= HALF)
            tidxb[sl] = jnp.where(oobb, HALF + (db & 7), db)

        def mbody(r, inner):
            for t in range(8):
                sl = pl.ds(t * 16, 16)
                w = hps[b][r, sl]
                lo = plsc.bitcast(w << 16, jnp.float32)
                hi = plsc.bitcast(w & jnp.int32(-65536), jnp.float32)
                gas[b][r, sl] = gas[b][r, sl] * lo
                gbs[b][r, sl] = gbs[b][r, sl] * hi
            return inner
        lax.fori_loop(0, CHUNK, mbody, 0)
        pltpu.sync_copy(gas[b], acc.at[tidxa], add=True)
        pltpu.sync_copy(gbs[b], acc.at[tidxb], add=True)

    fire(0, 0)
    fire(1, 1)

    def pair_body(j2, carry):
        for b in (0, 1):
            chunk = j2 * 2 + b
            wait_in(b)

            @pl.when(chunk + 2 < PCH)
            def _():
                fire(chunk + 2, b)
        return carry
    lax.fori_loop(0, PCH // 2, pair_body, 0)

    wait_in(0)

    plsc.subcore_barrier()
    for g in range(63):
        sz = CHUNK if g < 62 else HALF - 62 * CHUNK

        @pl.when(g % 16 == s)
        def _():
            pltpu.sync_copy(acc.at[pl.ds(g * CHUNK, sz)],
                            ga0.at[pl.ds(0, sz)])
            pltpu.sync_copy(ga0.at[pl.ds(0, sz)],
                            out_hbm.at[c, pl.ds(g * CHUNK, sz)])


@functools.cache
def _make_sc_gms():
    return pl.kernel(
        _sc_body,
        out_type=jax.ShapeDtypeStruct((2, HALF, D), jnp.float32),
        mesh=plsc.VectorSubcoreMesh(core_axis_name="c", subcore_axis_name="s"),
        compiler_params=pltpu.CompilerParams(needs_layout_passes=False),
        scratch_types=[
            pltpu.VMEM((T_EDGES,), jnp.int32),
            pltpu.VMEM((CHUNK,), jnp.int32),
            pltpu.VMEM((CHUNK,), jnp.int32),
            pltpu.VMEM((2 * CHUNK,), jnp.int32),
            pltpu.VMEM((2 * CHUNK,), jnp.int32),
            pltpu.VMEM((CHUNK, D), jnp.float32),
            pltpu.VMEM((CHUNK, D), jnp.float32),
            pltpu.VMEM((CHUNK, D), jnp.float32),
            pltpu.VMEM((CHUNK, D), jnp.float32),
            pltpu.VMEM((CHUNK, D), jnp.int32),
            pltpu.VMEM((CHUNK, D), jnp.int32),
            pltpu.VMEM_SHARED((ACC_ROWS, D), jnp.float32),
        ] + [pltpu.SemaphoreType.DMA] * 8,
    )


def _sc_gms(hv, he, src, dst):
    return _make_sc_gms()(hv, he, src, dst)


def kernel(node_inputs, edge_inputs, edge_index,
           Wn1, bn1, We1a, be1a, We1b, be1b, Wo1, bo1,
           Wn2, bn2, We2a, be2a, We2b, be2b, Wo2, bo2):
    src = edge_index[0]
    dst = edge_index[1]

    bn1_2 = bn1.reshape(1, D)
    be1a_2 = be1a.reshape(1, D)
    be1b_2 = be1b.reshape(1, D)
    bo1_2 = bo1.reshape(1, D)
    bn2_2 = bn2.reshape(1, D)
    be2a_2 = be2a.reshape(1, D)
    be2b_2 = be2b.reshape(1, D)
    bo2_2 = bo2.reshape(1, D)

    e_t = edge_inputs.T
    he1 = _edge_mlp(e_t, We1a, be1a_2, We1b, be1b_2)
    hv1 = _node_proj(node_inputs, Wn1, bn1_2)
    p1 = _sc_gms(hv1, he1, src, dst)
    he2 = _edge_mlp(e_t, We2a, be2a_2, We2b, be2b_2)
    hv2 = _mid_proj(p1, Wo1, bo1_2, Wn2, bn2_2)
    p2 = _sc_gms(hv2, he2, src, dst)
    return _fin_proj(p2, Wo2, bo2_2)
